# Initial kernel scaffold; baseline (speedup 1.0000x reference)
import jax
import jax.numpy as jnp
from jax.experimental import pallas as pl


def kernel(center_traj, traj_x, traj_edge_index, geo_x, geo_edge_index, traj_freq, geo_freq, tW0, tb0, tW1, tb1, tW2, tb2, gW0, gb0, gW1, gb1, gW2, gb2, in_w, in_b, out_w, out_b, lin_w, lin_b):
    raise NotImplementedError("write your pallas kernel here")


# speedup vs baseline: 6.5377x; 6.5377x over previous
import functools

import jax
import jax.numpy as jnp
from jax import lax
from jax.experimental import pallas as pl
from jax.experimental.pallas import tpu as pltpu
from jax.experimental.pallas import tpu_sc as plsc

N = 10000
B = 16
NPG = N // B
D = 128
E = 320000
L = 64
THR = 0.5

NP = 10240

NC = 2
NS = 16
NW = NC * NS
ECH = 128
EP = 327680
IB = 16
NIB = EP // NW // IB // ECH
RPT = NP // NS

DEG_CH = 128
DEG_EP = 2 * EP
DEG_NCH = DEG_EP // NW // DEG_CH
DPT = (2 * NP) // NS

BLK = 2048
NBLK = NP // BLK

_mesh = plsc.VectorSubcoreMesh(
    core_axis_name="c", subcore_axis_name="s", num_cores=NC, num_subcores=NS)


@functools.partial(
    pl.kernel,
    out_type=jax.ShapeDtypeStruct((NC * 2 * NP,), jnp.float32),
    mesh=_mesh,
    scratch_types=[
        pltpu.VMEM((DEG_NCH, DEG_CH), jnp.int32),
        pltpu.VMEM((128,), jnp.float32),
        pltpu.VMEM((DPT,), jnp.float32),
        pltpu.VMEM_SHARED((2 * NP,), jnp.float32),
    ],
)
def _deg_kernel(dst_hbm, out_hbm, dstv, onesv, zerov, acc):
    c = lax.axis_index("c")
    s = lax.axis_index("s")
    wid = c * NS + s
    pltpu.sync_copy(dst_hbm.at[wid], dstv)
    for k in range(8):
        onesv[pl.ds(k * 16, 16)] = jnp.ones((16,), jnp.float32)

    def zfill(k, carry):
        zerov[pl.ds(k * 16, 16)] = jnp.zeros((16,), jnp.float32)
        return carry
    lax.fori_loop(0, DPT // 16, zfill, 0)

    pltpu.sync_copy(zerov, acc.at[pl.ds(s * DPT, DPT)])
    plsc.subcore_barrier()

    def body(j, carry):
        pltpu.sync_copy(onesv, acc.at[dstv.at[j]], add=True)
        return carry
    lax.fori_loop(0, DEG_NCH, body, 0)
    plsc.subcore_barrier()

    pltpu.sync_copy(acc.at[pl.ds(s * DPT, DPT)],
                    out_hbm.at[pl.ds(c * (2 * NP) + s * DPT, DPT)])


@functools.partial(
    pl.kernel,
    out_type=jax.ShapeDtypeStruct((NC, NP, D), jnp.float32),
    mesh=_mesh,
    scratch_types=[
        pltpu.VMEM((IB, ECH), jnp.int32),
        pltpu.VMEM((IB, ECH), jnp.int32),
        pltpu.VMEM((2, ECH, D), jnp.float32),
        pltpu.VMEM_SHARED((NP, D), jnp.float32),
        pltpu.SemaphoreType.DMA,
    ],
)
def _scatter_kernel(g_hbm, src_hbm, dst_hbm, out_hbm, srcv, dstv, rows,
                    acc, sem):
    c = lax.axis_index("c")
    s = lax.axis_index("s")
    wid = c * NS + s

    @pl.when(c == 0)
    def _():
        pltpu.sync_copy(g_hbm.at[pl.ds(s * RPT, RPT)], acc.at[pl.ds(s * RPT, RPT)])

    @pl.when(c == 1)
    def _():
        def zb(i, carry):
            for k in range(D // 16):
                rows[0, i, pl.ds(k * 16, 16)] = jnp.zeros((16,), jnp.float32)
            return carry
        lax.fori_loop(0, ECH, zb, 0)
        for r in range(RPT // ECH):
            pltpu.sync_copy(rows.at[0], acc.at[pl.ds(s * RPT + r * ECH, ECH)])
    plsc.subcore_barrier()

    def blk(bi, carry):
        pltpu.sync_copy(src_hbm.at[wid, bi], srcv)
        pltpu.sync_copy(dst_hbm.at[wid, bi], dstv)
        pltpu.async_copy(g_hbm.at[srcv.at[0]], rows.at[0], sem)

        def inner(jj, carry2):
            for bb in range(2):
                j = jj * 2 + bb
                pltpu.make_async_copy(g_hbm.at[srcv.at[j]], rows.at[bb],
                                      sem).wait()

                @pl.when(j + 1 < IB)
                def _():
                    pltpu.async_copy(g_hbm.at[srcv.at[j + 1]], rows.at[1 - bb],
                                     sem)
                pltpu.sync_copy(rows.at[bb], acc.at[dstv.at[j]], add=True)
            return carry2
        lax.fori_loop(0, IB // 2, inner, 0)
        return carry
    lax.fori_loop(0, NIB, blk, 0)
    plsc.subcore_barrier()

    pltpu.sync_copy(acc.at[pl.ds(s * RPT, RPT)],
                    out_hbm.at[c, pl.ds(s * RPT, RPT)])


CPT = (B * L) // NW


@functools.partial(
    pl.kernel,
    out_type=jax.ShapeDtypeStruct((B * L, D), jnp.float32),
    mesh=_mesh,
    scratch_types=[
        pltpu.VMEM((CPT,), jnp.int32),
        pltpu.VMEM((CPT, D), jnp.float32),
        pltpu.SemaphoreType.DMA,
    ],
)
def _gather_kernel(x_hbm, idx_hbm, out_hbm, idxv, rows, sem):
    c = lax.axis_index("c")
    s = lax.axis_index("s")
    wid = c * NS + s
    pltpu.sync_copy(idx_hbm.at[pl.ds(wid * CPT, CPT)], idxv)
    pltpu.async_copy(x_hbm.at[idxv], rows, sem).wait()
    pltpu.sync_copy(rows, out_hbm.at[pl.ds(wid * CPT, CPT)])


def _dinv(dp0, dp1):
    return lax.rsqrt(dp0 + dp1 + 1.0)


def _matT(a, w):
    return lax.dot_general(a, w, (((1,), (1,)), ((), ())),
                           preferred_element_type=jnp.float32)


def _first_body(x_ref, w_ref, dp0_ref, dp1_ref, g_ref):
    di = _dinv(dp0_ref[...], dp1_ref[...])
    g_ref[...] = _matT(x_ref[...], w_ref[...]) * di


def _mid_body(p_ref, dp0_ref, dp1_ref, b_ref, w_ref, g_ref):
    di = _dinv(dp0_ref[...], dp1_ref[...])
    x = (p_ref[0, :, :] + p_ref[1, :, :]) * di + b_ref[...]
    g_ref[...] = _matT(x, w_ref[...]) * di


_row_spec = pl.BlockSpec((BLK, D), lambda i: (i, 0))
_p_spec = pl.BlockSpec((2, BLK, D), lambda i: (0, i, 0))
_dp_spec = pl.BlockSpec((BLK, 1), lambda i: (i, 0))
_w_spec = pl.BlockSpec((D, D), lambda i: (0, 0))
_b_spec = pl.BlockSpec((1, D), lambda i: (0, 0))

_first_tc = pl.pallas_call(
    _first_body, grid=(NBLK,),
    in_specs=[_row_spec, _w_spec, _dp_spec, _dp_spec],
    out_specs=_row_spec,
    out_shape=jax.ShapeDtypeStruct((NP, D), jnp.float32))

_mid_tc = pl.pallas_call(
    _mid_body, grid=(NBLK,),
    in_specs=[_p_spec, _dp_spec, _dp_spec, _b_spec, _w_spec],
    out_specs=_row_spec,
    out_shape=jax.ShapeDtypeStruct((NP, D), jnp.float32))


def _final_body(pt_ref, dpt0_ref, dpt1_ref, bt_ref, ft_ref,
                pg_ref, dpg0_ref, dpg1_ref, bg_ref, fg_ref,
                tout_ref, perst_ref, persg_ref,
                pt_acc, pg_acc, ct_acc, cg_acc):
    i = pl.program_id(0)

    @pl.when(i == 0)
    def _():
        pt_acc[...] = jnp.zeros_like(pt_acc)
        pg_acc[...] = jnp.zeros_like(pg_acc)
        ct_acc[...] = jnp.zeros_like(ct_acc)
        cg_acc[...] = jnp.zeros_like(cg_acc)

    dit = _dinv(dpt0_ref[...], dpt1_ref[...])
    xt = (pt_ref[0, :, :] + pt_ref[1, :, :]) * dit + bt_ref[...]
    tout_ref[...] = xt
    dig = _dinv(dpg0_ref[...], dpg1_ref[...])
    xg = (pg_ref[0, :, :] + pg_ref[1, :, :]) * dig + bg_ref[...]

    rows = i * BLK + lax.broadcasted_iota(jnp.int32, (B, BLK), 1)
    gidx = lax.broadcasted_iota(jnp.int32, (B, BLK), 0)
    member = jnp.where((rows >= gidx * NPG) & (rows < (gidx + 1) * NPG),
                       1.0, 0.0)
    mkt = jnp.where(ft_ref[...] >= THR, 1.0, 0.0)
    mkg = jnp.where(fg_ref[...] >= THR, 1.0, 0.0)
    pt_acc[...] += jnp.dot(member, xt * mkt, preferred_element_type=jnp.float32)
    pg_acc[...] += jnp.dot(member, xg * mkg, preferred_element_type=jnp.float32)
    ct_acc[...] += jnp.dot(member, jnp.broadcast_to(mkt, (BLK, D)),
                           preferred_element_type=jnp.float32)
    cg_acc[...] += jnp.dot(member, jnp.broadcast_to(mkg, (BLK, D)),
                           preferred_element_type=jnp.float32)

    @pl.when(i == NBLK - 1)
    def _():
        perst_ref[...] = pt_acc[...] / jnp.maximum(ct_acc[...], 1.0)
        persg_ref[...] = pg_acc[...] / jnp.maximum(cg_acc[...], 1.0)


_pers_spec = pl.BlockSpec((B, D), lambda i: (0, 0))

_final_tc = pl.pallas_call(
    _final_body, grid=(NBLK,),
    in_specs=[_p_spec, _dp_spec, _dp_spec, _b_spec, _dp_spec,
              _p_spec, _dp_spec, _dp_spec, _b_spec, _dp_spec],
    out_specs=[_row_spec, _pers_spec, _pers_spec],
    out_shape=[jax.ShapeDtypeStruct((NP, D), jnp.float32),
               jax.ShapeDtypeStruct((B, D), jnp.float32),
               jax.ShapeDtypeStruct((B, D), jnp.float32)],
    scratch_shapes=[pltpu.VMEM((B, D), jnp.float32),
                    pltpu.VMEM((B, D), jnp.float32),
                    pltpu.VMEM((B, D), jnp.float32),
                    pltpu.VMEM((B, D), jnp.float32)])


def _mha_body(pt_ref, pg_ref, wv_ref, bv_ref, ow_ref, ob_ref, lw_ref, lb_ref,
              out_ref):
    p = jnp.concatenate([pt_ref[...], pg_ref[...]], axis=1)
    v = _matT(p, wv_ref[...]) + bv_ref[...]
    o = _matT(v, ow_ref[...]) + ob_ref[...]
    out_ref[...] = _matT(o, lw_ref[...]) + lb_ref[...]


_mha_tc = pl.pallas_call(
    _mha_body,
    out_shape=jax.ShapeDtypeStruct((B, D), jnp.float32))


def kernel(center_traj, traj_x, traj_edge_index, geo_x, geo_edge_index,
           traj_freq, geo_freq, tW0, tb0, tW1, tb1, tW2, tb2,
           gW0, gb0, gW1, gb1, gW2, gb2,
           in_w, in_b, out_w, out_b, lin_w, lin_b):
    epad = EP - E
    src_t = jnp.pad(traj_edge_index[0], (0, epad)).reshape(NW, NIB, IB, ECH)
    dst_t = jnp.pad(traj_edge_index[1], (0, epad),
                    constant_values=N).reshape(NW, NIB, IB, ECH)
    src_g = jnp.pad(geo_edge_index[0], (0, epad)).reshape(NW, NIB, IB, ECH)
    dst_g = jnp.pad(geo_edge_index[1], (0, epad),
                    constant_values=N).reshape(NW, NIB, IB, ECH)
    dst_all = jnp.concatenate(
        [jnp.pad(traj_edge_index[1], (0, epad), constant_values=N),
         jnp.pad(geo_edge_index[1], (0, epad), constant_values=N) + NP]
    ).reshape(NW, DEG_NCH, DEG_CH)

    degp = _deg_kernel(dst_all).reshape(NC, 2, NP)
    dpt0 = degp[0, 0].reshape(NP, 1)
    dpt1 = degp[1, 0].reshape(NP, 1)
    dpg0 = degp[0, 1].reshape(NP, 1)
    dpg1 = degp[1, 1].reshape(NP, 1)

    xt0 = jnp.pad(traj_x, ((0, NP - N), (0, 0)))
    xg0 = jnp.pad(geo_x, ((0, NP - N), (0, 0)))

    bt0 = tb0.reshape(1, D)
    bt1 = tb1.reshape(1, D)
    bt2 = tb2.reshape(1, D)
    bg0 = gb0.reshape(1, D)
    bg1 = gb1.reshape(1, D)
    bg2 = gb2.reshape(1, D)

    g0t = _first_tc(xt0, tW0, dpt0, dpt1)
    g0g = _first_tc(xg0, gW0, dpg0, dpg1)
    s0t = _scatter_kernel(g0t, src_t, dst_t)
    s0g = _scatter_kernel(g0g, src_g, dst_g)
    g1t = _mid_tc(s0t, dpt0, dpt1, bt0, tW1)
    g1g = _mid_tc(s0g, dpg0, dpg1, bg0, gW1)
    s1t = _scatter_kernel(g1t, src_t, dst_t)
    s1g = _scatter_kernel(g1g, src_g, dst_g)
    g2t = _mid_tc(s1t, dpt0, dpt1, bt1, tW2)
    g2g = _mid_tc(s1g, dpg0, dpg1, bg1, gW2)
    s2t = _scatter_kernel(g2t, src_t, dst_t)
    s2g = _scatter_kernel(g2g, src_g, dst_g)

    ft = jnp.pad(traj_freq, (0, NP - N)).reshape(NP, 1)
    fg = jnp.pad(geo_freq, (0, NP - N)).reshape(NP, 1)
    tout, pers_t, pers_g = _final_tc(s2t, dpt0, dpt1, bt2, ft,
                                     s2g, dpg0, dpg1, bg2, fg)

    wv = in_w[4 * D:]
    bv = in_b[4 * D:].reshape(1, 2 * D)
    up = _mha_tc(pers_t, pers_g, wv, bv, out_w, out_b.reshape(1, 2 * D),
                 lin_w, lin_b.reshape(1, D))

    ptr = (jnp.arange(B, dtype=center_traj.dtype) * NPG)[:, None]
    cidx = (center_traj + ptr).reshape(B * L)
    ce = _gather_kernel(tout, cidx)
    return (ce.reshape(B, L, D), up.reshape(B, 1, D))


# speedup vs baseline: 6.9355x; 1.0608x over previous
import functools

import jax
import jax.numpy as jnp
from jax import lax
from jax.experimental import pallas as pl
from jax.experimental.pallas import tpu as pltpu
from jax.experimental.pallas import tpu_sc as plsc

N = 10000
B = 16
NPG = N // B
D = 128
E = 320000
L = 64
THR = 0.5

NP = 10240

NC = 2
NS = 16
NW = NC * NS
ECH = 128
EP = 327680
IB = 16
NIB = EP // NW // IB // ECH
RPT = NP // NS

DEG_CH = 128
DEG_EP = 2 * EP
DEG_NCH = DEG_EP // NW // DEG_CH
DPT = (2 * NP) // NS

BLK = 2048
NBLK = NP // BLK

_mesh = plsc.VectorSubcoreMesh(
    core_axis_name="c", subcore_axis_name="s", num_cores=NC, num_subcores=NS)


@functools.partial(
    pl.kernel,
    out_type=jax.ShapeDtypeStruct((NC * 2 * NP,), jnp.float32),
    mesh=_mesh,
    scratch_types=[
        pltpu.VMEM((DEG_NCH, DEG_CH), jnp.int32),
        pltpu.VMEM((128,), jnp.float32),
        pltpu.VMEM((DPT,), jnp.float32),
        pltpu.VMEM_SHARED((2 * NP,), jnp.float32),
    ],
)
def _deg_kernel(dst_hbm, out_hbm, dstv, onesv, zerov, acc):
    c = lax.axis_index("c")
    s = lax.axis_index("s")
    wid = c * NS + s
    pltpu.sync_copy(dst_hbm.at[wid], dstv)
    for k in range(8):
        onesv[pl.ds(k * 16, 16)] = jnp.ones((16,), jnp.float32)

    def zfill(k, carry):
        zerov[pl.ds(k * 16, 16)] = jnp.zeros((16,), jnp.float32)
        return carry
    lax.fori_loop(0, DPT // 16, zfill, 0)

    pltpu.sync_copy(zerov, acc.at[pl.ds(s * DPT, DPT)])
    plsc.subcore_barrier()

    def body(j, carry):
        pltpu.sync_copy(onesv, acc.at[dstv.at[j]], add=True)
        return carry
    lax.fori_loop(0, DEG_NCH, body, 0)
    plsc.subcore_barrier()

    pltpu.sync_copy(acc.at[pl.ds(s * DPT, DPT)],
                    out_hbm.at[pl.ds(c * (2 * NP) + s * DPT, DPT)])


@functools.partial(
    pl.kernel,
    out_type=jax.ShapeDtypeStruct((NC, NP, D), jnp.float32),
    mesh=_mesh,
    scratch_types=[
        pltpu.VMEM((IB, ECH), jnp.int32),
        pltpu.VMEM((IB, ECH), jnp.int32),
        pltpu.VMEM((2, ECH, D), jnp.float32),
        pltpu.VMEM_SHARED((NP, D), jnp.float32),
        pltpu.SemaphoreType.DMA,
    ],
)
def _scatter_kernel(g_hbm, src_hbm, dst_hbm, out_hbm, srcv, dstv, rows,
                    acc, sem):
    c = lax.axis_index("c")
    s = lax.axis_index("s")
    wid = c * NS + s

    @pl.when(c == 0)
    def _():
        pltpu.sync_copy(g_hbm.at[pl.ds(s * RPT, RPT)], acc.at[pl.ds(s * RPT, RPT)])

    @pl.when(c == 1)
    def _():
        def zb(i, carry):
            for k in range(D // 16):
                rows[0, i, pl.ds(k * 16, 16)] = jnp.zeros((16,), jnp.float32)
            return carry
        lax.fori_loop(0, ECH, zb, 0)
        for r in range(RPT // ECH):
            pltpu.sync_copy(rows.at[0], acc.at[pl.ds(s * RPT + r * ECH, ECH)])
    plsc.subcore_barrier()

    def blk(bi, carry):
        pltpu.sync_copy(src_hbm.at[wid, bi], srcv)
        pltpu.sync_copy(dst_hbm.at[wid, bi], dstv)
        pltpu.async_copy(g_hbm.at[srcv.at[0]], rows.at[0], sem)

        def inner(jj, carry2):
            for bb in range(2):
                j = jj * 2 + bb
                pltpu.make_async_copy(g_hbm.at[srcv.at[j]], rows.at[bb],
                                      sem).wait()

                @pl.when(j + 1 < IB)
                def _():
                    pltpu.async_copy(g_hbm.at[srcv.at[j + 1]], rows.at[1 - bb],
                                     sem)
                pltpu.sync_copy(rows.at[bb], acc.at[dstv.at[j]], add=True)
            return carry2
        lax.fori_loop(0, IB // 2, inner, 0)
        return carry
    lax.fori_loop(0, NIB, blk, 0)
    plsc.subcore_barrier()

    pltpu.sync_copy(acc.at[pl.ds(s * RPT, RPT)],
                    out_hbm.at[c, pl.ds(s * RPT, RPT)])


CPT = (B * L) // NW


@functools.partial(
    pl.kernel,
    out_type=jax.ShapeDtypeStruct((B * L, D), jnp.float32),
    mesh=_mesh,
    scratch_types=[
        pltpu.VMEM((CPT,), jnp.int32),
        pltpu.VMEM((CPT, D), jnp.float32),
        pltpu.SemaphoreType.DMA,
    ],
)
def _gather_kernel(x_hbm, idx_hbm, out_hbm, idxv, rows, sem):
    c = lax.axis_index("c")
    s = lax.axis_index("s")
    wid = c * NS + s
    pltpu.sync_copy(idx_hbm.at[pl.ds(wid * CPT, CPT)], idxv)
    pltpu.async_copy(x_hbm.at[idxv], rows, sem).wait()
    pltpu.sync_copy(rows, out_hbm.at[pl.ds(wid * CPT, CPT)])


def _dinv(dp0, dp1):
    return lax.rsqrt(dp0 + dp1 + 1.0)


def _matT(a, w):
    return lax.dot_general(a, w, (((1,), (1,)), ((), ())),
                           preferred_element_type=jnp.float32)


def _first_body(x_ref, w_ref, dp0_ref, dp1_ref, g_ref):
    di = _dinv(dp0_ref[...], dp1_ref[...])
    g_ref[...] = _matT(x_ref[...], w_ref[...]) * di


def _mid_body(p_ref, dp0_ref, dp1_ref, b_ref, w_ref, g_ref):
    di = _dinv(dp0_ref[...], dp1_ref[...])
    x = (p_ref[0, :, :] + p_ref[1, :, :]) * di + b_ref[...]
    g_ref[...] = _matT(x, w_ref[...]) * di


_row_spec = pl.BlockSpec((BLK, D), lambda i: (i, 0))
_p_spec = pl.BlockSpec((2, BLK, D), lambda i: (0, i, 0))
_dp_spec = pl.BlockSpec((BLK, 1), lambda i: (i, 0))
_w_spec = pl.BlockSpec((D, D), lambda i: (0, 0))
_b_spec = pl.BlockSpec((1, D), lambda i: (0, 0))

_first_tc = pl.pallas_call(
    _first_body, grid=(NBLK,),
    in_specs=[_row_spec, _w_spec, _dp_spec, _dp_spec],
    out_specs=_row_spec,
    out_shape=jax.ShapeDtypeStruct((NP, D), jnp.float32))

_mid_tc = pl.pallas_call(
    _mid_body, grid=(NBLK,),
    in_specs=[_p_spec, _dp_spec, _dp_spec, _b_spec, _w_spec],
    out_specs=_row_spec,
    out_shape=jax.ShapeDtypeStruct((NP, D), jnp.float32))


def _final_body(pt_ref, dpt0_ref, dpt1_ref, bt_ref, ft_ref,
                pg_ref, dpg0_ref, dpg1_ref, bg_ref, fg_ref,
                tout_ref, perst_ref, persg_ref,
                pt_acc, pg_acc, ct_acc, cg_acc):
    i = pl.program_id(0)

    @pl.when(i == 0)
    def _():
        pt_acc[...] = jnp.zeros_like(pt_acc)
        pg_acc[...] = jnp.zeros_like(pg_acc)
        ct_acc[...] = jnp.zeros_like(ct_acc)
        cg_acc[...] = jnp.zeros_like(cg_acc)

    dit = _dinv(dpt0_ref[...], dpt1_ref[...])
    xt = (pt_ref[0, :, :] + pt_ref[1, :, :]) * dit + bt_ref[...]
    tout_ref[...] = xt
    dig = _dinv(dpg0_ref[...], dpg1_ref[...])
    xg = (pg_ref[0, :, :] + pg_ref[1, :, :]) * dig + bg_ref[...]

    rows = i * BLK + lax.broadcasted_iota(jnp.int32, (B, BLK), 1)
    gidx = lax.broadcasted_iota(jnp.int32, (B, BLK), 0)
    member = jnp.where((rows >= gidx * NPG) & (rows < (gidx + 1) * NPG),
                       1.0, 0.0)
    mkt = jnp.where(ft_ref[...] >= THR, 1.0, 0.0)
    mkg = jnp.where(fg_ref[...] >= THR, 1.0, 0.0)
    pt_acc[...] += jnp.dot(member, xt * mkt, preferred_element_type=jnp.float32)
    pg_acc[...] += jnp.dot(member, xg * mkg, preferred_element_type=jnp.float32)
    ct_acc[...] += jnp.dot(member, jnp.broadcast_to(mkt, (BLK, D)),
                           preferred_element_type=jnp.float32)
    cg_acc[...] += jnp.dot(member, jnp.broadcast_to(mkg, (BLK, D)),
                           preferred_element_type=jnp.float32)

    @pl.when(i == NBLK - 1)
    def _():
        perst_ref[...] = pt_acc[...] / jnp.maximum(ct_acc[...], 1.0)
        persg_ref[...] = pg_acc[...] / jnp.maximum(cg_acc[...], 1.0)


_pers_spec = pl.BlockSpec((B, D), lambda i: (0, 0))

_final_tc = pl.pallas_call(
    _final_body, grid=(NBLK,),
    in_specs=[_p_spec, _dp_spec, _dp_spec, _b_spec, _dp_spec,
              _p_spec, _dp_spec, _dp_spec, _b_spec, _dp_spec],
    out_specs=[_row_spec, _pers_spec, _pers_spec],
    out_shape=[jax.ShapeDtypeStruct((NP, D), jnp.float32),
               jax.ShapeDtypeStruct((B, D), jnp.float32),
               jax.ShapeDtypeStruct((B, D), jnp.float32)],
    scratch_shapes=[pltpu.VMEM((B, D), jnp.float32),
                    pltpu.VMEM((B, D), jnp.float32),
                    pltpu.VMEM((B, D), jnp.float32),
                    pltpu.VMEM((B, D), jnp.float32)])


def _mha_body(pt_ref, pg_ref, wv_ref, bv_ref, ow_ref, ob_ref, lw_ref, lb_ref,
              out_ref):
    p = jnp.concatenate([pt_ref[...], pg_ref[...]], axis=1)
    v = _matT(p, wv_ref[...]) + bv_ref[...]
    o = _matT(v, ow_ref[...]) + ob_ref[...]
    out_ref[...] = _matT(o, lw_ref[...]) + lb_ref[...]


_mha_tc = pl.pallas_call(
    _mha_body,
    out_shape=jax.ShapeDtypeStruct((B, D), jnp.float32))


def kernel(center_traj, traj_x, traj_edge_index, geo_x, geo_edge_index,
           traj_freq, geo_freq, tW0, tb0, tW1, tb1, tW2, tb2,
           gW0, gb0, gW1, gb1, gW2, gb2,
           in_w, in_b, out_w, out_b, lin_w, lin_b):
    epw = E // NW
    padw = EP // NW - epw
    idt = center_traj.dtype
    pad_src = jnp.zeros((NW, padw), idt)
    pad_dst = jnp.broadcast_to(N + jnp.arange(padw, dtype=idt), (NW, padw))

    def _pad_pair(ei):
        s2 = jnp.concatenate([ei[0].reshape(NW, epw), pad_src], axis=1)
        d2 = jnp.concatenate([ei[1].reshape(NW, epw), pad_dst], axis=1)
        return (s2.reshape(NW, NIB, IB, ECH), d2.reshape(NW, NIB, IB, ECH))

    src_t, dst_t = _pad_pair(traj_edge_index)
    src_g, dst_g = _pad_pair(geo_edge_index)

    dpw = 2 * E // NW
    dpadw = DEG_EP // NW - dpw
    deg_pad = jnp.broadcast_to(
        N + jnp.arange(dpadw, dtype=idt) % (NP - N), (NW, dpadw))
    deg_real = jnp.concatenate(
        [traj_edge_index[1], geo_edge_index[1] + NP]).reshape(NW, dpw)
    dst_all = jnp.concatenate([deg_real, deg_pad], axis=1).reshape(
        NW, DEG_NCH, DEG_CH)

    degp = _deg_kernel(dst_all).reshape(NC, 2, NP)
    dpt0 = degp[0, 0].reshape(NP, 1)
    dpt1 = degp[1, 0].reshape(NP, 1)
    dpg0 = degp[0, 1].reshape(NP, 1)
    dpg1 = degp[1, 1].reshape(NP, 1)

    xt0 = jnp.pad(traj_x, ((0, NP - N), (0, 0)))
    xg0 = jnp.pad(geo_x, ((0, NP - N), (0, 0)))

    bt0 = tb0.reshape(1, D)
    bt1 = tb1.reshape(1, D)
    bt2 = tb2.reshape(1, D)
    bg0 = gb0.reshape(1, D)
    bg1 = gb1.reshape(1, D)
    bg2 = gb2.reshape(1, D)

    g0t = _first_tc(xt0, tW0, dpt0, dpt1)
    g0g = _first_tc(xg0, gW0, dpg0, dpg1)
    s0t = _scatter_kernel(g0t, src_t, dst_t)
    s0g = _scatter_kernel(g0g, src_g, dst_g)
    g1t = _mid_tc(s0t, dpt0, dpt1, bt0, tW1)
    g1g = _mid_tc(s0g, dpg0, dpg1, bg0, gW1)
    s1t = _scatter_kernel(g1t, src_t, dst_t)
    s1g = _scatter_kernel(g1g, src_g, dst_g)
    g2t = _mid_tc(s1t, dpt0, dpt1, bt1, tW2)
    g2g = _mid_tc(s1g, dpg0, dpg1, bg1, gW2)
    s2t = _scatter_kernel(g2t, src_t, dst_t)
    s2g = _scatter_kernel(g2g, src_g, dst_g)

    ft = jnp.pad(traj_freq, (0, NP - N)).reshape(NP, 1)
    fg = jnp.pad(geo_freq, (0, NP - N)).reshape(NP, 1)
    tout, pers_t, pers_g = _final_tc(s2t, dpt0, dpt1, bt2, ft,
                                     s2g, dpg0, dpg1, bg2, fg)

    wv = in_w[4 * D:]
    bv = in_b[4 * D:].reshape(1, 2 * D)
    up = _mha_tc(pers_t, pers_g, wv, bv, out_w, out_b.reshape(1, 2 * D),
                 lin_w, lin_b.reshape(1, D))

    ptr = (jnp.arange(B, dtype=center_traj.dtype) * NPG)[:, None]
    cidx = (center_traj + ptr).reshape(B * L)
    ce = _gather_kernel(tout, cidx)
    return (ce.reshape(B, L, D), up.reshape(B, 1, D))
